# Initial kernel scaffold; baseline (speedup 1.0000x reference)
import jax
import jax.numpy as jnp
from jax.experimental import pallas as pl


def kernel(x, embeddings):
    raise NotImplementedError("write your pallas kernel here")


# speedup vs baseline: 2.1848x; 2.1848x over previous
import functools

import jax
import jax.numpy as jnp
from jax import lax
from jax.experimental import pallas as pl
from jax.experimental.pallas import tpu as pltpu
from jax.experimental.pallas import tpu_sc as plsc

_BATCH = 4096
_INPUT_DIM = 100
_NUM_EMB = 1000
_EMB_DIM = 64
_V_MIN, _V_MAX = -6.0, 6.0
_SCALE = (_NUM_EMB - 1) / (_V_MAX - _V_MIN)

_NC, _NS = 2, 16
_NW = _NC * _NS
_N = _BATCH * _INPUT_DIM
_PW = _N // _NW
_GROUP = 16
_CHUNK = 256
_GPC = _CHUNK // _GROUP
_NCHUNKS = _PW // _CHUNK
_CHUNK_OUT = _CHUNK * _EMB_DIM


def _body(x_hbm, emb_hbm, out_hbm, x_v, table_v, obuf, sem):
    wid = lax.axis_index("s") * _NC + lax.axis_index("c")
    base = wid * _PW

    pltpu.sync_copy(emb_hbm, table_v)
    pltpu.sync_copy(x_hbm.at[pl.ds(base, _PW)], x_v)

    lane = lax.iota(jnp.int32, _GROUP)
    st0 = lane * _EMB_DIM

    def group_body(gg, c):
        goff = c * _CHUNK + gg * _GROUP
        xv = x_v[pl.ds(goff, _GROUP)]
        xs = jnp.clip((xv - _V_MIN) * _SCALE, 0.0, float(_NUM_EMB - 1))
        il = xs.astype(jnp.int32)
        ih = jnp.minimum(il + 1, _NUM_EMB - 1)
        wh = xs - il.astype(jnp.float32)
        wl = 1.0 - wh
        blo = il * _EMB_DIM
        bhi = ih * _EMB_DIM
        stb = st0 + gg * (_GROUP * _EMB_DIM)
        for d in range(_EMB_DIM):
            lo = plsc.load_gather(table_v, [blo + d])
            hi = plsc.load_gather(table_v, [bhi + d])
            plsc.store_scatter(obuf, [stb + d], wl * lo + wh * hi)
        return c

    def chunk_body(c, carry):
        lax.fori_loop(0, _GPC, group_body, c)
        pltpu.sync_copy(
            obuf, out_hbm.at[pl.ds(base * _EMB_DIM + c * _CHUNK_OUT, _CHUNK_OUT)]
        )
        return carry

    lax.fori_loop(0, _NCHUNKS, chunk_body, 0)


_sc_call = functools.partial(
    pl.kernel,
    out_type=jax.ShapeDtypeStruct((_N * _EMB_DIM,), jnp.float32),
    mesh=plsc.VectorSubcoreMesh(
        core_axis_name="c", subcore_axis_name="s", num_cores=_NC, num_subcores=_NS
    ),
    scratch_types=[
        pltpu.VMEM((_PW,), jnp.float32),
        pltpu.VMEM((_NUM_EMB * _EMB_DIM,), jnp.float32),
        pltpu.VMEM((_CHUNK_OUT,), jnp.float32),
        pltpu.SemaphoreType.DMA,
    ],
    compiler_params=pltpu.CompilerParams(needs_layout_passes=False),
)(_body)


def kernel(x, embeddings):
    out = _sc_call(x.reshape(-1), embeddings.reshape(-1))
    return out.reshape(_BATCH, _INPUT_DIM * _EMB_DIM)


# speedup vs baseline: 2.2271x; 1.0193x over previous
import functools

import jax
import jax.numpy as jnp
from jax import lax
from jax.experimental import pallas as pl
from jax.experimental.pallas import tpu as pltpu
from jax.experimental.pallas import tpu_sc as plsc

_BATCH = 4096
_INPUT_DIM = 100
_NUM_EMB = 1000
_EMB_DIM = 64
_V_MIN, _V_MAX = -6.0, 6.0
_SCALE = (_NUM_EMB - 1) / (_V_MAX - _V_MIN)

_NC, _NS = 2, 16
_NW = _NC * _NS
_N = _BATCH * _INPUT_DIM
_PW = _N // _NW
_GROUP = 16
_CHUNK = 256
_GPC = _CHUNK // _GROUP
_NCHUNKS = _PW // _CHUNK
_CHUNK_OUT = _CHUNK * _EMB_DIM
_NBUF = 2


def _body(x_hbm, emb_hbm, out_hbm, x_v, table_v, ob0, ob1, sem0, sem1):
    wid = lax.axis_index("s") * _NC + lax.axis_index("c")
    base = wid * _PW
    obufs = (ob0, ob1)
    sems = (sem0, sem1)

    pltpu.sync_copy(emb_hbm, table_v)
    pltpu.sync_copy(x_hbm.at[pl.ds(base, _PW)], x_v)

    lane = lax.iota(jnp.int32, _GROUP)
    st0 = lane * _EMB_DIM

    def fill(c, obuf):
        def group_body(gg, carry):
            goff = c * _CHUNK + gg * _GROUP
            xv = x_v[pl.ds(goff, _GROUP)]
            xs = jnp.clip((xv - _V_MIN) * _SCALE, 0.0, float(_NUM_EMB - 1))
            il = xs.astype(jnp.int32)
            ih = jnp.minimum(il + 1, _NUM_EMB - 1)
            wh = xs - il.astype(jnp.float32)
            wl = 1.0 - wh
            blo = il * _EMB_DIM
            bhi = ih * _EMB_DIM
            stb = st0 + gg * (_GROUP * _EMB_DIM)
            for d in range(_EMB_DIM):
                lo = plsc.load_gather(table_v, [blo + d])
                hi = plsc.load_gather(table_v, [bhi + d])
                plsc.store_scatter(obuf, [stb + d], wl * lo + wh * hi)
            return carry

        lax.fori_loop(0, _GPC, group_body, 0)

    def chunk_pair(ci, carry):
        for b in range(_NBUF):
            c = ci * _NBUF + b

            @pl.when(ci > 0)
            def _():
                pltpu.make_async_copy(
                    out_hbm.at[pl.ds(0, _CHUNK_OUT)], obufs[b], sems[b]
                ).wait()

            fill(c, obufs[b])
            pltpu.async_copy(
                obufs[b],
                out_hbm.at[pl.ds(base * _EMB_DIM + c * _CHUNK_OUT, _CHUNK_OUT)],
                sems[b],
            )
        return carry

    lax.fori_loop(0, _NCHUNKS // _NBUF, chunk_pair, 0)
    for b in range(_NBUF):
        pltpu.make_async_copy(
            out_hbm.at[pl.ds(0, _CHUNK_OUT)], obufs[b], sems[b]
        ).wait()


_sc_call = functools.partial(
    pl.kernel,
    out_type=jax.ShapeDtypeStruct((_N * _EMB_DIM,), jnp.float32),
    mesh=plsc.VectorSubcoreMesh(
        core_axis_name="c", subcore_axis_name="s", num_cores=_NC, num_subcores=_NS
    ),
    scratch_types=[
        pltpu.VMEM((_PW,), jnp.float32),
        pltpu.VMEM((_NUM_EMB * _EMB_DIM,), jnp.float32),
        pltpu.VMEM((_CHUNK_OUT,), jnp.float32),
        pltpu.VMEM((_CHUNK_OUT,), jnp.float32),
        pltpu.SemaphoreType.DMA,
        pltpu.SemaphoreType.DMA,
    ],
    compiler_params=pltpu.CompilerParams(needs_layout_passes=False),
)(_body)


def kernel(x, embeddings):
    out = _sc_call(x.reshape(-1), embeddings.reshape(-1))
    return out.reshape(_BATCH, _INPUT_DIM * _EMB_DIM)
